# Initial kernel scaffold; baseline (speedup 1.0000x reference)
import jax
import jax.numpy as jnp
from jax.experimental import pallas as pl


def kernel(x, conv_w, gn_gamma, gn_beta):
    raise NotImplementedError("write your pallas kernel here")


# speedup vs baseline: 4.9248x; 4.9248x over previous
import functools

import jax
import jax.numpy as jnp
from jax import lax
from jax.experimental import pallas as pl
from jax.experimental.pallas import tpu as pltpu
from jax.experimental.pallas import tpu_sc as plsc

KNN = 80
NEG = 0.2
EPSV = 1e-5
CO = 64
NB = 256


def _proj_body(x_ref, w1_ref, wd_ref, a_ref, b_ref):
    xf = x_ref[...]
    a_ref[...] = jnp.dot(xf, w1_ref[...], preferred_element_type=jnp.float32)
    b_ref[...] = jnp.dot(xf, wd_ref[...], preferred_element_type=jnp.float32)


def _thresh_body(x_ref, xb_ref, skey_ref, t_ref, i_ref):
    xt = x_ref[0]
    xb = xb_ref[0]
    n = xt.shape[1]
    inner = -2.0 * jnp.dot(xb.T, xt, preferred_element_type=jnp.float32)
    xx = jnp.sum(xt * xt, axis=0)[None, :]
    xxb = jnp.sum(xb * xb, axis=0)[:, None]
    d = (-xxb) - inner - xx

    bits = lax.bitcast_convert_type(d, jnp.int32)
    skey = bits ^ ((bits >> 31) & jnp.int32(0x7FFFFFFF))
    skey_ref[0] = skey

    cnt0 = jnp.sum((skey >= 0).astype(jnp.int32), axis=1, keepdims=True)
    t = jnp.where(cnt0 >= KNN, jnp.int32(0), jnp.int32(-2147483648))
    t = jnp.broadcast_to(t, (NB, 1))
    for bit in range(30, -1, -1):
        cand = t + jnp.int32(1 << bit)
        cnt = jnp.sum((skey >= cand).astype(jnp.int32), axis=1, keepdims=True)
        t = jnp.where(cnt >= KNN, cand, t)

    c_gt = jnp.sum((skey > t).astype(jnp.int32), axis=1, keepdims=True)
    r = KNN - c_gt

    eq = skey == t
    idxv = lax.broadcasted_iota(jnp.int32, (NB, n), 1)
    acc = jnp.zeros((NB, 1), jnp.int32)
    for bit in range(10, -1, -1):
        cand = acc + jnp.int32(1 << bit)
        cle = jnp.sum((eq & (idxv < cand)).astype(jnp.int32), axis=1, keepdims=True)
        acc = jnp.where(cle < r, cand, acc)

    t_ref[0, 0] = t[:, 0]
    i_ref[0, 0] = acc[:, 0]


def _sc_body(skey_hbm, t_hbm, i_hbm, a_hbm, out_hbm,
             skrow, idxbuf, rows, outrow, tbuf, ibuf):
    nc = 2
    nw = 32
    total = skey_hbm.shape[0]
    rows_per_w = total // nw
    wid = lax.axis_index("s") * nc + lax.axis_index("c")

    for z in range(6):
        idxbuf[pl.ds(z * 16, 16)] = jnp.zeros((16,), jnp.int32)

    lane = lax.iota(jnp.int32, 16)

    ones = jnp.full((16,), 1, jnp.int32)
    zeros = jnp.full((16,), 0, jnp.int32)
    trash = jnp.full((16,), 95, jnp.int32)

    def chunk_body(blk, _):
        base = wid * rows_per_w + blk * 128
        pltpu.sync_copy(t_hbm.at[pl.ds(base, 128)], tbuf.at[pl.ds(0, 128)])
        pltpu.sync_copy(i_hbm.at[pl.ds(base, 128)], ibuf.at[pl.ds(0, 128)])

        def row_body(j, _2):
            r = base + j
            bb = (r // 2048) * 2048
            pltpu.sync_copy(skey_hbm.at[r], skrow)
            tv = jnp.full((16,), tbuf[pl.ds(j, 16)][0], jnp.int32)
            iv = jnp.full((16,), ibuf[pl.ds(j, 16)][0], jnp.int32)

            bbv = jnp.full((16,), bb, jnp.int32)

            def comp_body(c, off):
                kv = skrow[pl.ds(c * 16, 16)]
                midx = lane + jnp.full((16,), c * 16, jnp.int32)
                inc = (kv > tv) | ((kv == tv) & (midx <= iv))
                inci = jnp.where(inc, ones, zeros)
                cs = plsc.cumsum(inci)
                offv = jnp.full((16,), off, jnp.int32)
                pos = jnp.where(inc, offv + cs - 1, trash)
                plsc.store_scatter(idxbuf, [pos], midx + bbv)
                return off + jnp.sum(inci)

            lax.fori_loop(0, 128, comp_body, jnp.int32(0))

            pltpu.sync_copy(a_hbm.at[idxbuf], rows)

            for ch in range(4):
                v0 = rows[0, pl.ds(ch * 16, 16)]

                def red_body(k, carry):
                    s1, s2, mx, mn = carry
                    v = rows[k, pl.ds(ch * 16, 16)]
                    return (s1 + v, s2 + v * v,
                            jnp.maximum(mx, v), jnp.minimum(mn, v))

                s1, s2, mx, mn = lax.fori_loop(
                    1, KNN, red_body, (v0, v0 * v0, v0, v0))
                outrow[pl.ds(ch * 16, 16)] = s1
                outrow[pl.ds(64 + ch * 16, 16)] = s2
                outrow[pl.ds(128 + ch * 16, 16)] = mx
                outrow[pl.ds(192 + ch * 16, 16)] = mn

            pltpu.sync_copy(outrow, out_hbm.at[r])
            return 0

        lax.fori_loop(0, 128, row_body, 0)
        return 0

    lax.fori_loop(0, rows_per_w // 128, chunk_body, 0)


def _final_body(st_ref, b_ref, g_ref, be_ref, o_ref):
    st = st_ref[...]
    bv = b_ref[...]
    s1 = st[:, 0:64]
    s2 = st[:, 64:128]
    mx = st[:, 128:192]
    mn = st[:, 192:256]
    kf = jnp.float32(KNN)
    sum_y = s1 + kf * bv
    sum_y2 = s2 + 2.0 * bv * s1 + kf * bv * bv
    n = bv.shape[0]
    cnt = jnp.float32(32 * n * KNN)
    gs0 = jnp.sum(sum_y[:, 0:32])
    gs1 = jnp.sum(sum_y[:, 32:64])
    gq0 = jnp.sum(sum_y2[:, 0:32])
    gq1 = jnp.sum(sum_y2[:, 32:64])
    m0 = gs0 / cnt
    m1 = gs1 / cnt
    v0 = gq0 / cnt - m0 * m0
    v1 = gq1 / cnt - m1 * m1
    r0 = lax.rsqrt(v0 + EPSV)
    r1 = lax.rsqrt(v1 + EPSV)
    ch = lax.broadcasted_iota(jnp.int32, (1, CO), 1)
    meanc = jnp.where(ch < 32, m0, m1)
    rstdc = jnp.where(ch < 32, r0, r1)
    gam = g_ref[...]
    bet = be_ref[...]
    ysel = jnp.where(gam >= 0, mx, mn) + bv
    z = gam * (ysel - meanc) * rstdc + bet
    o_ref[...] = jnp.where(z > 0, z, NEG * z)


def kernel(x, conv_w, gn_gamma, gn_beta):
    B, N, CI = x.shape
    BN = B * N
    w1t = jnp.transpose(conv_w[:, :CI])
    wdt = jnp.transpose(conv_w[:, CI:] - conv_w[:, :CI])

    xf = x.reshape(BN, CI)
    a_arr, b_arr = pl.pallas_call(
        _proj_body,
        out_shape=(jax.ShapeDtypeStruct((BN, CO), jnp.float32),
                   jax.ShapeDtypeStruct((BN, CO), jnp.float32)),
    )(xf, w1t, wdt)

    xc = jnp.transpose(x, (0, 2, 1))
    nblk = N // NB
    skey, tstar, istar = pl.pallas_call(
        _thresh_body,
        grid=(B, nblk),
        in_specs=[
            pl.BlockSpec((1, CI, N), lambda b, i: (b, 0, 0)),
            pl.BlockSpec((1, CI, NB), lambda b, i: (b, 0, i)),
        ],
        out_specs=(
            pl.BlockSpec((1, NB, N), lambda b, i: (b, i, 0)),
            pl.BlockSpec((1, 1, NB), lambda b, i: (b * nblk + i, 0, 0)),
            pl.BlockSpec((1, 1, NB), lambda b, i: (b * nblk + i, 0, 0)),
        ),
        out_shape=(
            jax.ShapeDtypeStruct((B, N, N), jnp.int32),
            jax.ShapeDtypeStruct((B * nblk, 1, NB), jnp.int32),
            jax.ShapeDtypeStruct((B * nblk, 1, NB), jnp.int32),
        ),
    )(xc, xc)

    skey2 = skey.reshape(BN, N)
    tflat = tstar.reshape(BN)
    iflat = istar.reshape(BN)

    mesh = plsc.VectorSubcoreMesh(core_axis_name="c", subcore_axis_name="s")
    stats = pl.kernel(
        _sc_body,
        mesh=mesh,
        compiler_params=pltpu.CompilerParams(
            needs_layout_passes=False, use_tc_tiling_on_sc=False),
        out_type=jax.ShapeDtypeStruct((BN, 4 * CO), jnp.float32),
        scratch_types=[
            pltpu.VMEM((N,), jnp.int32),
            pltpu.VMEM((96,), jnp.int32),
            pltpu.VMEM((96, CO), jnp.float32),
            pltpu.VMEM((4 * CO,), jnp.float32),
            pltpu.VMEM((144,), jnp.int32),
            pltpu.VMEM((144,), jnp.int32),
        ],
    )(skey2, tflat, iflat, a_arr)

    out = pl.pallas_call(
        _final_body,
        grid=(B,),
        in_specs=[
            pl.BlockSpec((N, 4 * CO), lambda b: (b, 0)),
            pl.BlockSpec((N, CO), lambda b: (b, 0)),
            pl.BlockSpec((1, CO), lambda b: (0, 0)),
            pl.BlockSpec((1, CO), lambda b: (0, 0)),
        ],
        out_specs=pl.BlockSpec((N, CO), lambda b: (b, 0)),
        out_shape=jax.ShapeDtypeStruct((BN, CO), jnp.float32),
    )(stats.reshape(BN, 4 * CO), b_arr, gn_gamma[None, :], gn_beta[None, :])

    return out.reshape(B, N, CO)


# speedup vs baseline: 14.3827x; 2.9205x over previous
import functools

import jax
import jax.numpy as jnp
from jax import lax
from jax.experimental import pallas as pl
from jax.experimental.pallas import tpu as pltpu
from jax.experimental.pallas import tpu_sc as plsc

KNN = 80
NEG = 0.2
EPSV = 1e-5
CO = 64
NB = 256


def _proj_body(x_ref, w1_ref, wd_ref, a_ref, b_ref):
    xf = x_ref[...]
    a_ref[...] = jnp.dot(xf, w1_ref[...], preferred_element_type=jnp.float32)
    b_ref[...] = jnp.dot(xf, wd_ref[...], preferred_element_type=jnp.float32)


def _thresh_body(x_ref, xb_ref, skey_ref, t_ref, i_ref):
    xt = x_ref[0]
    xb = xb_ref[0]
    n = xt.shape[1]
    inner = -2.0 * jnp.dot(xb.T, xt, preferred_element_type=jnp.float32)
    xx = jnp.sum(xt * xt, axis=0)[None, :]
    xxb = jnp.sum(xb * xb, axis=0)[:, None]
    d = (-xxb) - inner - xx

    bits = lax.bitcast_convert_type(d, jnp.int32)
    skey = bits ^ ((bits >> 31) & jnp.int32(0x7FFFFFFF))
    skey_ref[0] = skey

    cnt0 = jnp.sum((skey >= 0).astype(jnp.int32), axis=1, keepdims=True)
    t = jnp.where(cnt0 >= KNN, jnp.int32(0), jnp.int32(-2147483648))
    t = jnp.broadcast_to(t, (NB, 1))
    for bit in range(30, -1, -1):
        cand = t + jnp.int32(1 << bit)
        cnt = jnp.sum((skey >= cand).astype(jnp.int32), axis=1, keepdims=True)
        t = jnp.where(cnt >= KNN, cand, t)

    c_gt = jnp.sum((skey > t).astype(jnp.int32), axis=1, keepdims=True)
    r = KNN - c_gt

    eq = skey == t
    idxv = lax.broadcasted_iota(jnp.int32, (NB, n), 1)
    acc = jnp.zeros((NB, 1), jnp.int32)
    for bit in range(10, -1, -1):
        cand = acc + jnp.int32(1 << bit)
        cle = jnp.sum((eq & (idxv < cand)).astype(jnp.int32), axis=1, keepdims=True)
        acc = jnp.where(cle < r, cand, acc)

    t_ref[0, 0] = t[:, 0]
    i_ref[0, 0] = acc[:, 0]


def _sc_body(skey_hbm, t_hbm, i_hbm, a_hbm, out_hbm,
             sk0, sk1, idx0, idx1, rows0, rows1, outbuf, tbuf, ibuf,
             ssem0, ssem1, gsem0, gsem1):
    nc = 2
    nw = 32
    total = skey_hbm.shape[0]
    rows_per_w = total // nw
    wid = lax.axis_index("s") * nc + lax.axis_index("c")

    lane = lax.iota(jnp.int32, 16)
    ones = jnp.full((16,), 1, jnp.int32)
    zeros = jnp.full((16,), 0, jnp.int32)

    sks = (sk0, sk1)
    idxs = (idx0, idx1)
    rowss = (rows0, rows1)
    ssems = (ssem0, ssem1)
    gsems = (gsem0, gsem1)

    def compact(j, bbv, sk, idx):
        tv = jnp.full((16,), tbuf[pl.ds(j, 16)][0], jnp.int32)
        iv = jnp.full((16,), ibuf[pl.ds(j, 16)][0], jnp.int32)

        def comp_body(c, off):
            kv = sk[pl.ds(c * 16, 16)]
            midx = lane + jnp.full((16,), c * 16, jnp.int32)
            inc = (kv > tv) | ((kv == tv) & (midx <= iv))
            inci = jnp.where(inc, ones, zeros)
            cs = plsc.cumsum(inci)
            plsc.store_scatter(idx, [off + cs - 1], midx + bbv, mask=inc)
            return off + plsc.all_reduce_population_count(inc)

        lax.fori_loop(0, 128, comp_body, zeros, unroll=4)

    def reduce_to(j, rows):
        v0 = (rows[0, pl.ds(0, 16)], rows[0, pl.ds(16, 16)],
              rows[0, pl.ds(32, 16)], rows[0, pl.ds(48, 16)])

        def red_body(k, carry):
            out = []
            for ch in range(4):
                s1, s2, mx, mn = carry[ch]
                v = rows[k, pl.ds(ch * 16, 16)]
                out.append((s1 + v, s2 + v * v,
                            jnp.maximum(mx, v), jnp.minimum(mn, v)))
            return tuple(out)

        acc = lax.fori_loop(1, KNN, red_body,
                            tuple((v, v * v, v, v) for v in v0), unroll=2)
        for ch in range(4):
            s1, s2, mx, mn = acc[ch]
            outbuf[j, pl.ds(ch * 16, 16)] = s1
            outbuf[j, pl.ds(64 + ch * 16, 16)] = s2
            outbuf[j, pl.ds(128 + ch * 16, 16)] = mx
            outbuf[j, pl.ds(192 + ch * 16, 16)] = mn

    def chunk_body(blk, _):
        base = wid * rows_per_w + blk * 128
        bbv = jnp.full((16,), base // 2048 * 2048, jnp.int32)
        pltpu.sync_copy(t_hbm.at[pl.ds(base, 128)], tbuf.at[pl.ds(0, 128)])
        pltpu.sync_copy(i_hbm.at[pl.ds(base, 128)], ibuf.at[pl.ds(0, 128)])
        pltpu.async_copy(skey_hbm.at[base], sk0, ssem0)

        def two_rows(tt, _2):
            for p in range(2):
                j = 2 * tt + p
                r = base + j
                sk, idx, rows = sks[p], idxs[p], rowss[p]
                rn = jnp.minimum(r + 1, total - 1)
                pltpu.async_copy(skey_hbm.at[rn], sks[1 - p], ssems[1 - p])
                pltpu.make_async_copy(skey_hbm.at[r], sk, ssems[p]).wait()
                compact(j, bbv, sk, idx)
                pltpu.async_copy(a_hbm.at[idx], rows, gsems[p])
                prev = 1 - p
                if p == 1:
                    pltpu.make_async_copy(
                        a_hbm.at[idxs[prev]], rowss[prev], gsems[prev]).wait()
                    reduce_to(j - 1, rowss[prev])
                else:
                    @pl.when(tt > 0)
                    def _():
                        pltpu.make_async_copy(
                            a_hbm.at[idxs[prev]], rowss[prev],
                            gsems[prev]).wait()
                        reduce_to(j - 1, rowss[prev])
            return 0

        lax.fori_loop(0, 64, two_rows, 0)
        pltpu.make_async_copy(a_hbm.at[idx1], rows1, gsem1).wait()
        reduce_to(127, rows1)
        pltpu.make_async_copy(skey_hbm.at[base], sk0, ssem0).wait()
        pltpu.sync_copy(outbuf, out_hbm.at[pl.ds(base, 128)])
        return 0

    lax.fori_loop(0, rows_per_w // 128, chunk_body, 0)


def _final_body(st_ref, b_ref, g_ref, be_ref, o_ref):
    st = st_ref[...]
    bv = b_ref[...]
    s1 = st[:, 0:64]
    s2 = st[:, 64:128]
    mx = st[:, 128:192]
    mn = st[:, 192:256]
    kf = jnp.float32(KNN)
    sum_y = s1 + kf * bv
    sum_y2 = s2 + 2.0 * bv * s1 + kf * bv * bv
    n = bv.shape[0]
    cnt = jnp.float32(32 * n * KNN)
    gs0 = jnp.sum(sum_y[:, 0:32])
    gs1 = jnp.sum(sum_y[:, 32:64])
    gq0 = jnp.sum(sum_y2[:, 0:32])
    gq1 = jnp.sum(sum_y2[:, 32:64])
    m0 = gs0 / cnt
    m1 = gs1 / cnt
    v0 = gq0 / cnt - m0 * m0
    v1 = gq1 / cnt - m1 * m1
    r0 = lax.rsqrt(v0 + EPSV)
    r1 = lax.rsqrt(v1 + EPSV)
    ch = lax.broadcasted_iota(jnp.int32, (1, CO), 1)
    meanc = jnp.where(ch < 32, m0, m1)
    rstdc = jnp.where(ch < 32, r0, r1)
    gam = g_ref[...]
    bet = be_ref[...]
    ysel = jnp.where(gam >= 0, mx, mn) + bv
    z = gam * (ysel - meanc) * rstdc + bet
    o_ref[...] = jnp.where(z > 0, z, NEG * z)


def kernel(x, conv_w, gn_gamma, gn_beta):
    B, N, CI = x.shape
    BN = B * N
    w1t = jnp.transpose(conv_w[:, :CI])
    wdt = jnp.transpose(conv_w[:, CI:] - conv_w[:, :CI])

    xf = x.reshape(BN, CI)
    a_arr, b_arr = pl.pallas_call(
        _proj_body,
        out_shape=(jax.ShapeDtypeStruct((BN, CO), jnp.float32),
                   jax.ShapeDtypeStruct((BN, CO), jnp.float32)),
    )(xf, w1t, wdt)

    xc = jnp.transpose(x, (0, 2, 1))
    nblk = N // NB
    skey, tstar, istar = pl.pallas_call(
        _thresh_body,
        grid=(B, nblk),
        in_specs=[
            pl.BlockSpec((1, CI, N), lambda b, i: (b, 0, 0)),
            pl.BlockSpec((1, CI, NB), lambda b, i: (b, 0, i)),
        ],
        out_specs=(
            pl.BlockSpec((1, NB, N), lambda b, i: (b, i, 0)),
            pl.BlockSpec((1, 1, NB), lambda b, i: (b * nblk + i, 0, 0)),
            pl.BlockSpec((1, 1, NB), lambda b, i: (b * nblk + i, 0, 0)),
        ),
        out_shape=(
            jax.ShapeDtypeStruct((B, N, N), jnp.int32),
            jax.ShapeDtypeStruct((B * nblk, 1, NB), jnp.int32),
            jax.ShapeDtypeStruct((B * nblk, 1, NB), jnp.int32),
        ),
    )(xc, xc)

    skey2 = skey.reshape(BN, N)
    tflat = tstar.reshape(BN)
    iflat = istar.reshape(BN)

    mesh = plsc.VectorSubcoreMesh(core_axis_name="c", subcore_axis_name="s")
    stats = pl.kernel(
        _sc_body,
        mesh=mesh,
        compiler_params=pltpu.CompilerParams(
            needs_layout_passes=False, use_tc_tiling_on_sc=False),
        out_type=jax.ShapeDtypeStruct((BN, 4 * CO), jnp.float32),
        scratch_types=[
            pltpu.VMEM((N,), jnp.int32),
            pltpu.VMEM((N,), jnp.int32),
            pltpu.VMEM((KNN,), jnp.int32),
            pltpu.VMEM((KNN,), jnp.int32),
            pltpu.VMEM((KNN, CO), jnp.float32),
            pltpu.VMEM((KNN, CO), jnp.float32),
            pltpu.VMEM((128, 4 * CO), jnp.float32),
            pltpu.VMEM((144,), jnp.int32),
            pltpu.VMEM((144,), jnp.int32),
            pltpu.SemaphoreType.DMA,
            pltpu.SemaphoreType.DMA,
            pltpu.SemaphoreType.DMA,
            pltpu.SemaphoreType.DMA,
        ],
    )(skey2, tflat, iflat, a_arr)

    out = pl.pallas_call(
        _final_body,
        grid=(B,),
        in_specs=[
            pl.BlockSpec((N, 4 * CO), lambda b: (b, 0)),
            pl.BlockSpec((N, CO), lambda b: (b, 0)),
            pl.BlockSpec((1, CO), lambda b: (0, 0)),
            pl.BlockSpec((1, CO), lambda b: (0, 0)),
        ],
        out_specs=pl.BlockSpec((N, CO), lambda b: (b, 0)),
        out_shape=jax.ShapeDtypeStruct((BN, CO), jnp.float32),
    )(stats.reshape(BN, 4 * CO), b_arr, gn_gamma[None, :], gn_beta[None, :])

    return out.reshape(B, N, CO)


# speedup vs baseline: 15.9700x; 1.1104x over previous
import functools

import jax
import jax.numpy as jnp
from jax import lax
from jax.experimental import pallas as pl
from jax.experimental.pallas import tpu as pltpu
from jax.experimental.pallas import tpu_sc as plsc

KNN = 80
NEG = 0.2
EPSV = 1e-5
CO = 64
NB = 256


def _proj_body(x_ref, w1_ref, wd_ref, a_ref, b_ref):
    xf = x_ref[...]
    a_ref[...] = jnp.dot(xf, w1_ref[...], preferred_element_type=jnp.float32)
    b_ref[...] = jnp.dot(xf, wd_ref[...], preferred_element_type=jnp.float32)


def _thresh_body(x_ref, xb_ref, skey_ref, t_ref):
    xt = x_ref[0]
    xb = xb_ref[0]
    n = xt.shape[1]
    inner = -2.0 * jnp.dot(xb.T, xt, preferred_element_type=jnp.float32)
    xx = jnp.sum(xt * xt, axis=0)[None, :]
    xxb = jnp.sum(xb * xb, axis=0)[:, None]
    d = (-xxb) - inner - xx

    bits = lax.bitcast_convert_type(d, jnp.int32)
    skey = bits ^ ((bits >> 31) & jnp.int32(0x7FFFFFFF))
    skey_ref[0] = skey

    cnt0 = jnp.sum((skey >= 0).astype(jnp.int32), axis=1, keepdims=True)
    t = jnp.where(cnt0 >= KNN, jnp.int32(0), jnp.int32(-2147483648))
    t = jnp.broadcast_to(t, (NB, 1))
    for bit in range(30, -1, -1):
        cand = t + jnp.int32(1 << bit)
        cnt = jnp.sum((skey >= cand).astype(jnp.int32), axis=1, keepdims=True)
        t = jnp.where(cnt >= KNN, cand, t)

    t_ref[0, 0] = t[:, 0]


def _sc_body(skey_hbm, t_hbm, a_hbm, out_hbm,
             sk0, sk1, idx0, idx1, rows0, rows1, outbuf, tbuf,
             ssem0, ssem1, gsem0, gsem1):
    nc = 2
    nw = 32
    total = skey_hbm.shape[0]
    rows_per_w = total // nw
    wid = lax.axis_index("s") * nc + lax.axis_index("c")

    lane = lax.iota(jnp.int32, 16)
    ones = jnp.full((16,), 1, jnp.int32)
    zeros = jnp.full((16,), 0, jnp.int32)

    sks = (sk0, sk1)
    idxs = (idx0, idx1)
    rowss = (rows0, rows1)
    ssems = (ssem0, ssem1)
    gsems = (gsem0, gsem1)

    kcap = jnp.full((16,), KNN, jnp.int32)

    def compact(j, bbv, sk, idx):
        tv = jnp.full((16,), tbuf[pl.ds(j, 16)][0], jnp.int32)

        def comp_body(c, off):
            kv = sk[pl.ds(c * 16, 16)]
            midx = lane + jnp.full((16,), c * 16, jnp.int32)
            inc = kv >= tv
            inci = jnp.where(inc, ones, zeros)
            pos = off + plsc.cumsum(inci) - 1
            plsc.store_scatter(idx, [pos], midx + bbv,
                               mask=inc & (pos < kcap))
            return off + plsc.all_reduce_population_count(inc)

        lax.fori_loop(0, 128, comp_body, zeros, unroll=4)

    def reduce_to(j, rows):
        v0 = (rows[0, pl.ds(0, 16)], rows[0, pl.ds(16, 16)],
              rows[0, pl.ds(32, 16)], rows[0, pl.ds(48, 16)])

        def red_body(k, carry):
            out = []
            for ch in range(4):
                s1, s2, mx, mn = carry[ch]
                v = rows[k, pl.ds(ch * 16, 16)]
                out.append((s1 + v, s2 + v * v,
                            jnp.maximum(mx, v), jnp.minimum(mn, v)))
            return tuple(out)

        acc = lax.fori_loop(1, KNN, red_body,
                            tuple((v, v * v, v, v) for v in v0), unroll=2)
        for ch in range(4):
            s1, s2, mx, mn = acc[ch]
            outbuf[j, pl.ds(ch * 16, 16)] = s1
            outbuf[j, pl.ds(64 + ch * 16, 16)] = s2
            outbuf[j, pl.ds(128 + ch * 16, 16)] = mx
            outbuf[j, pl.ds(192 + ch * 16, 16)] = mn

    def chunk_body(blk, _):
        base = wid * rows_per_w + blk * 128
        bbv = jnp.full((16,), base // 2048 * 2048, jnp.int32)
        pltpu.sync_copy(t_hbm.at[pl.ds(base, 128)], tbuf.at[pl.ds(0, 128)])
        pltpu.async_copy(skey_hbm.at[base], sk0, ssem0)

        def two_rows(tt, _2):
            for p in range(2):
                j = 2 * tt + p
                r = base + j
                sk, idx, rows = sks[p], idxs[p], rowss[p]
                rn = jnp.minimum(r + 1, total - 1)
                pltpu.async_copy(skey_hbm.at[rn], sks[1 - p], ssems[1 - p])
                pltpu.make_async_copy(skey_hbm.at[r], sk, ssems[p]).wait()
                compact(j, bbv, sk, idx)
                pltpu.async_copy(a_hbm.at[idx], rows, gsems[p])
                prev = 1 - p
                if p == 1:
                    pltpu.make_async_copy(
                        a_hbm.at[idxs[prev]], rowss[prev], gsems[prev]).wait()
                    reduce_to(j - 1, rowss[prev])
                else:
                    @pl.when(tt > 0)
                    def _():
                        pltpu.make_async_copy(
                            a_hbm.at[idxs[prev]], rowss[prev],
                            gsems[prev]).wait()
                        reduce_to(j - 1, rowss[prev])
            return 0

        lax.fori_loop(0, 64, two_rows, 0)
        pltpu.make_async_copy(a_hbm.at[idx1], rows1, gsem1).wait()
        reduce_to(127, rows1)
        pltpu.make_async_copy(skey_hbm.at[base], sk0, ssem0).wait()
        pltpu.sync_copy(outbuf, out_hbm.at[pl.ds(base, 128)])
        return 0

    lax.fori_loop(0, rows_per_w // 128, chunk_body, 0)


def _final_body(st_ref, b_ref, g_ref, be_ref, o_ref):
    st = st_ref[...]
    bv = b_ref[...]
    s1 = st[:, 0:64]
    s2 = st[:, 64:128]
    mx = st[:, 128:192]
    mn = st[:, 192:256]
    kf = jnp.float32(KNN)
    sum_y = s1 + kf * bv
    sum_y2 = s2 + 2.0 * bv * s1 + kf * bv * bv
    n = bv.shape[0]
    cnt = jnp.float32(32 * n * KNN)
    gs0 = jnp.sum(sum_y[:, 0:32])
    gs1 = jnp.sum(sum_y[:, 32:64])
    gq0 = jnp.sum(sum_y2[:, 0:32])
    gq1 = jnp.sum(sum_y2[:, 32:64])
    m0 = gs0 / cnt
    m1 = gs1 / cnt
    v0 = gq0 / cnt - m0 * m0
    v1 = gq1 / cnt - m1 * m1
    r0 = lax.rsqrt(v0 + EPSV)
    r1 = lax.rsqrt(v1 + EPSV)
    ch = lax.broadcasted_iota(jnp.int32, (1, CO), 1)
    meanc = jnp.where(ch < 32, m0, m1)
    rstdc = jnp.where(ch < 32, r0, r1)
    gam = g_ref[...]
    bet = be_ref[...]
    ysel = jnp.where(gam >= 0, mx, mn) + bv
    z = gam * (ysel - meanc) * rstdc + bet
    o_ref[...] = jnp.where(z > 0, z, NEG * z)


def kernel(x, conv_w, gn_gamma, gn_beta):
    B, N, CI = x.shape
    BN = B * N
    w1t = jnp.transpose(conv_w[:, :CI])
    wdt = jnp.transpose(conv_w[:, CI:] - conv_w[:, :CI])

    xf = x.reshape(BN, CI)
    a_arr, b_arr = pl.pallas_call(
        _proj_body,
        out_shape=(jax.ShapeDtypeStruct((BN, CO), jnp.float32),
                   jax.ShapeDtypeStruct((BN, CO), jnp.float32)),
    )(xf, w1t, wdt)

    xc = jnp.transpose(x, (0, 2, 1))
    nblk = N // NB
    skey, tstar = pl.pallas_call(
        _thresh_body,
        grid=(B, nblk),
        in_specs=[
            pl.BlockSpec((1, CI, N), lambda b, i: (b, 0, 0)),
            pl.BlockSpec((1, CI, NB), lambda b, i: (b, 0, i)),
        ],
        out_specs=(
            pl.BlockSpec((1, NB, N), lambda b, i: (b, i, 0)),
            pl.BlockSpec((1, 1, NB), lambda b, i: (b * nblk + i, 0, 0)),
        ),
        out_shape=(
            jax.ShapeDtypeStruct((B, N, N), jnp.int32),
            jax.ShapeDtypeStruct((B * nblk, 1, NB), jnp.int32),
        ),
    )(xc, xc)

    skey2 = skey.reshape(BN, N)
    tflat = tstar.reshape(BN)

    mesh = plsc.VectorSubcoreMesh(core_axis_name="c", subcore_axis_name="s")
    stats = pl.kernel(
        _sc_body,
        mesh=mesh,
        compiler_params=pltpu.CompilerParams(
            needs_layout_passes=False, use_tc_tiling_on_sc=False),
        out_type=jax.ShapeDtypeStruct((BN, 4 * CO), jnp.float32),
        scratch_types=[
            pltpu.VMEM((N,), jnp.int32),
            pltpu.VMEM((N,), jnp.int32),
            pltpu.VMEM((KNN,), jnp.int32),
            pltpu.VMEM((KNN,), jnp.int32),
            pltpu.VMEM((KNN, CO), jnp.float32),
            pltpu.VMEM((KNN, CO), jnp.float32),
            pltpu.VMEM((128, 4 * CO), jnp.float32),
            pltpu.VMEM((144,), jnp.int32),
            pltpu.SemaphoreType.DMA,
            pltpu.SemaphoreType.DMA,
            pltpu.SemaphoreType.DMA,
            pltpu.SemaphoreType.DMA,
        ],
    )(skey2, tflat, a_arr)

    out = pl.pallas_call(
        _final_body,
        grid=(B,),
        in_specs=[
            pl.BlockSpec((N, 4 * CO), lambda b: (b, 0)),
            pl.BlockSpec((N, CO), lambda b: (b, 0)),
            pl.BlockSpec((1, CO), lambda b: (0, 0)),
            pl.BlockSpec((1, CO), lambda b: (0, 0)),
        ],
        out_specs=pl.BlockSpec((N, CO), lambda b: (b, 0)),
        out_shape=jax.ShapeDtypeStruct((BN, CO), jnp.float32),
    )(stats.reshape(BN, 4 * CO), b_arr, gn_gamma[None, :], gn_beta[None, :])

    return out.reshape(B, N, CO)


# speedup vs baseline: 16.1670x; 1.0123x over previous
import functools

import jax
import jax.numpy as jnp
from jax import lax
from jax.experimental import pallas as pl
from jax.experimental.pallas import tpu as pltpu
from jax.experimental.pallas import tpu_sc as plsc

KNN = 80
NEG = 0.2
EPSV = 1e-5
CO = 64
NB = 256


def _proj_body(x_ref, w1_ref, wd_ref, a_ref, b_ref):
    xf = x_ref[...]
    a_ref[...] = jnp.dot(xf, w1_ref[...], preferred_element_type=jnp.float32)
    b_ref[...] = jnp.dot(xf, wd_ref[...], preferred_element_type=jnp.float32)


def _thresh_body(x_ref, xb_ref, skey_ref, t_ref):
    xt = x_ref[0]
    xb = xb_ref[0]
    n = xt.shape[1]
    inner = -2.0 * jnp.dot(xb.T, xt, preferred_element_type=jnp.float32)
    xx = jnp.sum(xt * xt, axis=0)[None, :]
    xxb = jnp.sum(xb * xb, axis=0)[:, None]
    d = (-xxb) - inner - xx

    bits = lax.bitcast_convert_type(d, jnp.int32)
    skey = bits ^ ((bits >> 31) & jnp.int32(0x7FFFFFFF))
    skey_ref[0] = skey

    cnt0 = jnp.sum((skey >= 0).astype(jnp.int32), axis=1, keepdims=True)
    t = jnp.where(cnt0 >= KNN, jnp.int32(0), jnp.int32(-2147483648))
    t = jnp.broadcast_to(t, (NB, 1))
    for bit in range(30, -1, -1):
        cand = t + jnp.int32(1 << bit)
        cnt = jnp.sum((skey >= cand).astype(jnp.int32), axis=1, keepdims=True)
        t = jnp.where(cnt >= KNN, cand, t)

    t_ref[0, 0] = t[:, 0]


def _sc_body(skey_hbm, t_hbm, bb_hbm, a_hbm, out_hbm,
             sk0, sk1, idx0, idx1, rows0, rows1, outbuf, tbuf, bbuf,
             ssem0, ssem1, gsem0, gsem1):
    nc = 2
    nw = 32
    total = skey_hbm.shape[0]
    rows_per_w = total // nw
    ch = min(128, rows_per_w)
    wid = lax.axis_index("s") * nc + lax.axis_index("c")

    lane = lax.iota(jnp.int32, 16)
    ones = jnp.full((16,), 1, jnp.int32)
    zeros = jnp.full((16,), 0, jnp.int32)

    sks = (sk0, sk1)
    idxs = (idx0, idx1)
    rowss = (rows0, rows1)
    ssems = (ssem0, ssem1)
    gsems = (gsem0, gsem1)

    kcap = jnp.full((16,), KNN, jnp.int32)

    def compact(j, bbv, sk, idx):
        tv = jnp.full((16,), tbuf[pl.ds(j, 16)][0], jnp.int32)

        def comp_body(c, off):
            kv = sk[pl.ds(c * 16, 16)]
            midx = lane + jnp.full((16,), c * 16, jnp.int32)
            inc = kv >= tv
            inci = jnp.where(inc, ones, zeros)
            pos = off + plsc.cumsum(inci) - 1
            plsc.store_scatter(idx, [pos], midx + bbv,
                               mask=inc & (pos < kcap))
            return off + plsc.all_reduce_population_count(inc)

        lax.fori_loop(0, 128, comp_body, zeros, unroll=4)

    def reduce_to(j, rows):
        v0 = (rows[0, pl.ds(0, 16)], rows[0, pl.ds(16, 16)],
              rows[0, pl.ds(32, 16)], rows[0, pl.ds(48, 16)])

        def red_body(k, carry):
            out = []
            for ch in range(4):
                s1, s2, mx, mn = carry[ch]
                v = rows[k, pl.ds(ch * 16, 16)]
                out.append((s1 + v, s2 + v * v,
                            jnp.maximum(mx, v), jnp.minimum(mn, v)))
            return tuple(out)

        acc = lax.fori_loop(1, KNN, red_body,
                            tuple((v, v * v, v, v) for v in v0), unroll=2)
        for ch in range(4):
            s1, s2, mx, mn = acc[ch]
            outbuf[j, pl.ds(ch * 16, 16)] = s1
            outbuf[j, pl.ds(64 + ch * 16, 16)] = s2
            outbuf[j, pl.ds(128 + ch * 16, 16)] = mx
            outbuf[j, pl.ds(192 + ch * 16, 16)] = mn

    def chunk_body(blk, _):
        base = wid * rows_per_w + blk * ch
        pltpu.sync_copy(bb_hbm.at[pl.ds(0, 8)], bbuf.at[pl.ds(0, 8)])
        bbv = jnp.full((16,), bbuf[pl.ds(0, 16)][0], jnp.int32)
        pltpu.sync_copy(t_hbm.at[pl.ds(base, ch)], tbuf.at[pl.ds(0, ch)])
        pltpu.async_copy(skey_hbm.at[base], sk0, ssem0)

        def two_rows(tt, _2):
            for p in range(2):
                j = 2 * tt + p
                r = base + j
                sk, idx, rows = sks[p], idxs[p], rowss[p]
                rn = jnp.minimum(r + 1, total - 1)
                pltpu.async_copy(skey_hbm.at[rn], sks[1 - p], ssems[1 - p])
                pltpu.make_async_copy(skey_hbm.at[r], sk, ssems[p]).wait()
                compact(j, bbv, sk, idx)
                pltpu.async_copy(a_hbm.at[idx], rows, gsems[p])
                prev = 1 - p
                if p == 1:
                    pltpu.make_async_copy(
                        a_hbm.at[idxs[prev]], rowss[prev], gsems[prev]).wait()
                    reduce_to(j - 1, rowss[prev])
                else:
                    @pl.when(tt > 0)
                    def _():
                        pltpu.make_async_copy(
                            a_hbm.at[idxs[prev]], rowss[prev],
                            gsems[prev]).wait()
                        reduce_to(j - 1, rowss[prev])
            return 0

        lax.fori_loop(0, ch // 2, two_rows, 0)
        pltpu.make_async_copy(a_hbm.at[idx1], rows1, gsem1).wait()
        reduce_to(ch - 1, rows1)
        pltpu.make_async_copy(skey_hbm.at[base], sk0, ssem0).wait()
        pltpu.sync_copy(outbuf, out_hbm.at[pl.ds(base, ch)])
        return 0

    lax.fori_loop(0, rows_per_w // ch, chunk_body, 0)


def _final_body(st_ref, b_ref, g_ref, be_ref, o_ref):
    st = st_ref[...]
    bv = b_ref[...]
    s1 = st[:, 0:64]
    s2 = st[:, 64:128]
    mx = st[:, 128:192]
    mn = st[:, 192:256]
    kf = jnp.float32(KNN)
    sum_y = s1 + kf * bv
    sum_y2 = s2 + 2.0 * bv * s1 + kf * bv * bv
    n = bv.shape[0]
    cnt = jnp.float32(32 * n * KNN)
    gs0 = jnp.sum(sum_y[:, 0:32])
    gs1 = jnp.sum(sum_y[:, 32:64])
    gq0 = jnp.sum(sum_y2[:, 0:32])
    gq1 = jnp.sum(sum_y2[:, 32:64])
    m0 = gs0 / cnt
    m1 = gs1 / cnt
    v0 = gq0 / cnt - m0 * m0
    v1 = gq1 / cnt - m1 * m1
    r0 = lax.rsqrt(v0 + EPSV)
    r1 = lax.rsqrt(v1 + EPSV)
    ch = lax.broadcasted_iota(jnp.int32, (1, CO), 1)
    meanc = jnp.where(ch < 32, m0, m1)
    rstdc = jnp.where(ch < 32, r0, r1)
    gam = g_ref[...]
    bet = be_ref[...]
    ysel = jnp.where(gam >= 0, mx, mn) + bv
    z = gam * (ysel - meanc) * rstdc + bet
    o_ref[...] = jnp.where(z > 0, z, NEG * z)


def kernel(x, conv_w, gn_gamma, gn_beta):
    B, N, CI = x.shape
    BN = B * N
    w1t = jnp.transpose(conv_w[:, :CI])
    wdt = jnp.transpose(conv_w[:, CI:] - conv_w[:, :CI])

    xf = x.reshape(BN, CI)
    a_arr, b_arr = pl.pallas_call(
        _proj_body,
        out_shape=(jax.ShapeDtypeStruct((BN, CO), jnp.float32),
                   jax.ShapeDtypeStruct((BN, CO), jnp.float32)),
    )(xf, w1t, wdt)

    xc = jnp.transpose(x, (0, 2, 1))
    nblk = N // NB
    thresh_call = pl.pallas_call(
        _thresh_body,
        grid=(1, nblk),
        in_specs=[
            pl.BlockSpec((1, CI, N), lambda b, i: (0, 0, 0)),
            pl.BlockSpec((1, CI, NB), lambda b, i: (0, 0, i)),
        ],
        out_specs=(
            pl.BlockSpec((1, NB, N), lambda b, i: (i, 0, 0)),
            pl.BlockSpec((1, 1, NB), lambda b, i: (i, 0, 0)),
        ),
        out_shape=(
            jax.ShapeDtypeStruct((nblk, NB, N), jnp.int32),
            jax.ShapeDtypeStruct((nblk, 1, NB), jnp.int32),
        ),
    )

    mesh = plsc.VectorSubcoreMesh(core_axis_name="c", subcore_axis_name="s")
    rows_per_w = N // 32
    chb = min(128, rows_per_w)
    sc_call = pl.kernel(
        _sc_body,
        mesh=mesh,
        compiler_params=pltpu.CompilerParams(
            needs_layout_passes=False, use_tc_tiling_on_sc=False),
        out_type=jax.ShapeDtypeStruct((N, 4 * CO), jnp.float32),
        scratch_types=[
            pltpu.VMEM((N,), jnp.int32),
            pltpu.VMEM((N,), jnp.int32),
            pltpu.VMEM((KNN,), jnp.int32),
            pltpu.VMEM((KNN,), jnp.int32),
            pltpu.VMEM((KNN, CO), jnp.float32),
            pltpu.VMEM((KNN, CO), jnp.float32),
            pltpu.VMEM((chb, 4 * CO), jnp.float32),
            pltpu.VMEM((144,), jnp.int32),
            pltpu.VMEM((16,), jnp.int32),
            pltpu.SemaphoreType.DMA,
            pltpu.SemaphoreType.DMA,
            pltpu.SemaphoreType.DMA,
            pltpu.SemaphoreType.DMA,
        ],
    )

    stats_parts = []
    for b in range(B):
        skey_b, t_b = thresh_call(xc[b:b + 1], xc[b:b + 1])
        bb_b = jnp.full((8,), b * N, jnp.int32)
        stats_parts.append(
            sc_call(skey_b.reshape(N, N), t_b.reshape(N), bb_b, a_arr))
    stats = jnp.concatenate(stats_parts, axis=0)

    out = pl.pallas_call(
        _final_body,
        grid=(B,),
        in_specs=[
            pl.BlockSpec((N, 4 * CO), lambda b: (b, 0)),
            pl.BlockSpec((N, CO), lambda b: (b, 0)),
            pl.BlockSpec((1, CO), lambda b: (0, 0)),
            pl.BlockSpec((1, CO), lambda b: (0, 0)),
        ],
        out_specs=pl.BlockSpec((N, CO), lambda b: (b, 0)),
        out_shape=jax.ShapeDtypeStruct((BN, CO), jnp.float32),
    )(stats.reshape(BN, 4 * CO), b_arr, gn_gamma[None, :], gn_beta[None, :])

    return out.reshape(B, N, CO)


# speedup vs baseline: 16.1679x; 1.0001x over previous
import functools

import jax
import jax.numpy as jnp
from jax import lax
from jax.experimental import pallas as pl
from jax.experimental.pallas import tpu as pltpu
from jax.experimental.pallas import tpu_sc as plsc

KNN = 80
NEG = 0.2
EPSV = 1e-5
CO = 64
NB = 256


def _proj_body(x_ref, w1_ref, wd_ref, a_ref, b_ref):
    xf = x_ref[...]
    a_ref[...] = jnp.dot(xf, w1_ref[...], preferred_element_type=jnp.float32)
    b_ref[...] = jnp.dot(xf, wd_ref[...], preferred_element_type=jnp.float32)


def _thresh_body(x_ref, xb_ref, skey_ref, t_ref):
    xt = x_ref[0]
    xb = xb_ref[0]
    n = xt.shape[1]
    inner = -2.0 * jnp.dot(xb.T, xt, preferred_element_type=jnp.float32)
    xx = jnp.sum(xt * xt, axis=0)[None, :]
    xxb = jnp.sum(xb * xb, axis=0)[:, None]
    d = (-xxb) - inner - xx

    bits = lax.bitcast_convert_type(d, jnp.int32)
    skey = bits ^ ((bits >> 31) & jnp.int32(0x7FFFFFFF))
    skey_ref[0] = skey

    cnt0 = jnp.sum((skey >= 0).astype(jnp.int32), axis=1, keepdims=True)
    t = jnp.where(cnt0 >= KNN, jnp.int32(0), jnp.int32(-2147483648))
    t = jnp.broadcast_to(t, (NB, 1))
    for bit in range(30, -1, -1):
        cand = t + jnp.int32(1 << bit)
        cnt = jnp.sum((skey >= cand).astype(jnp.int32), axis=1, keepdims=True)
        t = jnp.where(cnt >= KNN, cand, t)

    t_ref[0, 0] = t[:, 0]


def _sc_body(skey_hbm, t_hbm, bb_hbm, a_hbm, out_hbm,
             sk0, sk1, idx0, idx1, rows0, rows1, outbuf, tbuf, bbuf,
             ssem0, ssem1, gsem0, gsem1):
    nc = 2
    nw = 32
    total = skey_hbm.shape[0]
    rows_per_w = total // nw
    ch = min(128, rows_per_w)
    wid = lax.axis_index("s") * nc + lax.axis_index("c")

    lane = lax.iota(jnp.int32, 16)
    ones = jnp.full((16,), 1, jnp.int32)
    zeros = jnp.full((16,), 0, jnp.int32)

    sks = (sk0, sk1)
    idxs = (idx0, idx1)
    rowss = (rows0, rows1)
    ssems = (ssem0, ssem1)
    gsems = (gsem0, gsem1)

    kcap = jnp.full((16,), KNN, jnp.int32)

    def compact(j, bbv, sk, idx):
        tv = jnp.full((16,), tbuf[pl.ds(j, 16)][0], jnp.int32)

        def comp_body(c, off):
            kv = sk[pl.ds(c * 16, 16)]
            midx = lane + jnp.full((16,), c * 16, jnp.int32)
            inc = kv >= tv
            inci = jnp.where(inc, ones, zeros)
            pos = off + plsc.cumsum(inci) - 1
            plsc.store_scatter(idx, [pos], midx + bbv,
                               mask=inc & (pos < kcap))
            return off + plsc.all_reduce_population_count(inc)

        lax.fori_loop(0, 128, comp_body, zeros, unroll=8)

    def reduce_to(j, rows):
        v0 = (rows[0, pl.ds(0, 16)], rows[0, pl.ds(16, 16)],
              rows[0, pl.ds(32, 16)], rows[0, pl.ds(48, 16)])

        def red_body(k, carry):
            out = []
            for ch in range(4):
                s1, s2, mx, mn = carry[ch]
                v = rows[k, pl.ds(ch * 16, 16)]
                out.append((s1 + v, s2 + v * v,
                            jnp.maximum(mx, v), jnp.minimum(mn, v)))
            return tuple(out)

        acc = lax.fori_loop(1, KNN, red_body,
                            tuple((v, v * v, v, v) for v in v0), unroll=4)
        for ch in range(4):
            s1, s2, mx, mn = acc[ch]
            outbuf[j, pl.ds(ch * 16, 16)] = s1
            outbuf[j, pl.ds(64 + ch * 16, 16)] = s2
            outbuf[j, pl.ds(128 + ch * 16, 16)] = mx
            outbuf[j, pl.ds(192 + ch * 16, 16)] = mn

    def chunk_body(blk, _):
        base = wid * rows_per_w + blk * ch
        pltpu.sync_copy(bb_hbm.at[pl.ds(0, 8)], bbuf.at[pl.ds(0, 8)])
        bbv = jnp.full((16,), bbuf[pl.ds(0, 16)][0], jnp.int32)
        pltpu.sync_copy(t_hbm.at[pl.ds(base, ch)], tbuf.at[pl.ds(0, ch)])
        pltpu.async_copy(skey_hbm.at[base], sk0, ssem0)

        def two_rows(tt, _2):
            for p in range(2):
                j = 2 * tt + p
                r = base + j
                sk, idx, rows = sks[p], idxs[p], rowss[p]
                rn = jnp.minimum(r + 1, total - 1)
                pltpu.async_copy(skey_hbm.at[rn], sks[1 - p], ssems[1 - p])
                pltpu.make_async_copy(skey_hbm.at[r], sk, ssems[p]).wait()
                compact(j, bbv, sk, idx)
                pltpu.async_copy(a_hbm.at[idx], rows, gsems[p])
                prev = 1 - p
                if p == 1:
                    pltpu.make_async_copy(
                        a_hbm.at[idxs[prev]], rowss[prev], gsems[prev]).wait()
                    reduce_to(j - 1, rowss[prev])
                else:
                    @pl.when(tt > 0)
                    def _():
                        pltpu.make_async_copy(
                            a_hbm.at[idxs[prev]], rowss[prev],
                            gsems[prev]).wait()
                        reduce_to(j - 1, rowss[prev])
            return 0

        lax.fori_loop(0, ch // 2, two_rows, 0)
        pltpu.make_async_copy(a_hbm.at[idx1], rows1, gsem1).wait()
        reduce_to(ch - 1, rows1)
        pltpu.make_async_copy(skey_hbm.at[base], sk0, ssem0).wait()
        pltpu.sync_copy(outbuf, out_hbm.at[pl.ds(base, ch)])
        return 0

    lax.fori_loop(0, rows_per_w // ch, chunk_body, 0)


def _final_body(st_ref, b_ref, g_ref, be_ref, o_ref):
    st = st_ref[...]
    bv = b_ref[...]
    s1 = st[:, 0:64]
    s2 = st[:, 64:128]
    mx = st[:, 128:192]
    mn = st[:, 192:256]
    kf = jnp.float32(KNN)
    sum_y = s1 + kf * bv
    sum_y2 = s2 + 2.0 * bv * s1 + kf * bv * bv
    n = bv.shape[0]
    cnt = jnp.float32(32 * n * KNN)
    gs0 = jnp.sum(sum_y[:, 0:32])
    gs1 = jnp.sum(sum_y[:, 32:64])
    gq0 = jnp.sum(sum_y2[:, 0:32])
    gq1 = jnp.sum(sum_y2[:, 32:64])
    m0 = gs0 / cnt
    m1 = gs1 / cnt
    v0 = gq0 / cnt - m0 * m0
    v1 = gq1 / cnt - m1 * m1
    r0 = lax.rsqrt(v0 + EPSV)
    r1 = lax.rsqrt(v1 + EPSV)
    ch = lax.broadcasted_iota(jnp.int32, (1, CO), 1)
    meanc = jnp.where(ch < 32, m0, m1)
    rstdc = jnp.where(ch < 32, r0, r1)
    gam = g_ref[...]
    bet = be_ref[...]
    ysel = jnp.where(gam >= 0, mx, mn) + bv
    z = gam * (ysel - meanc) * rstdc + bet
    o_ref[...] = jnp.where(z > 0, z, NEG * z)


def kernel(x, conv_w, gn_gamma, gn_beta):
    B, N, CI = x.shape
    BN = B * N
    w1t = jnp.transpose(conv_w[:, :CI])
    wdt = jnp.transpose(conv_w[:, CI:] - conv_w[:, :CI])

    xf = x.reshape(BN, CI)
    a_arr, b_arr = pl.pallas_call(
        _proj_body,
        out_shape=(jax.ShapeDtypeStruct((BN, CO), jnp.float32),
                   jax.ShapeDtypeStruct((BN, CO), jnp.float32)),
    )(xf, w1t, wdt)

    xc = jnp.transpose(x, (0, 2, 1))
    nblk = N // NB
    thresh_call = pl.pallas_call(
        _thresh_body,
        grid=(1, nblk),
        in_specs=[
            pl.BlockSpec((1, CI, N), lambda b, i: (0, 0, 0)),
            pl.BlockSpec((1, CI, NB), lambda b, i: (0, 0, i)),
        ],
        out_specs=(
            pl.BlockSpec((1, NB, N), lambda b, i: (i, 0, 0)),
            pl.BlockSpec((1, 1, NB), lambda b, i: (i, 0, 0)),
        ),
        out_shape=(
            jax.ShapeDtypeStruct((nblk, NB, N), jnp.int32),
            jax.ShapeDtypeStruct((nblk, 1, NB), jnp.int32),
        ),
    )

    mesh = plsc.VectorSubcoreMesh(core_axis_name="c", subcore_axis_name="s")
    rows_per_w = N // 32
    chb = min(128, rows_per_w)
    sc_call = pl.kernel(
        _sc_body,
        mesh=mesh,
        compiler_params=pltpu.CompilerParams(
            needs_layout_passes=False, use_tc_tiling_on_sc=False),
        out_type=jax.ShapeDtypeStruct((N, 4 * CO), jnp.float32),
        scratch_types=[
            pltpu.VMEM((N,), jnp.int32),
            pltpu.VMEM((N,), jnp.int32),
            pltpu.VMEM((KNN,), jnp.int32),
            pltpu.VMEM((KNN,), jnp.int32),
            pltpu.VMEM((KNN, CO), jnp.float32),
            pltpu.VMEM((KNN, CO), jnp.float32),
            pltpu.VMEM((chb, 4 * CO), jnp.float32),
            pltpu.VMEM((144,), jnp.int32),
            pltpu.VMEM((16,), jnp.int32),
            pltpu.SemaphoreType.DMA,
            pltpu.SemaphoreType.DMA,
            pltpu.SemaphoreType.DMA,
            pltpu.SemaphoreType.DMA,
        ],
    )

    stats_parts = []
    for b in range(B):
        skey_b, t_b = thresh_call(xc[b:b + 1], xc[b:b + 1])
        bb_b = jnp.full((8,), b * N, jnp.int32)
        stats_parts.append(
            sc_call(skey_b.reshape(N, N), t_b.reshape(N), bb_b, a_arr))
    stats = jnp.concatenate(stats_parts, axis=0)

    out = pl.pallas_call(
        _final_body,
        grid=(B,),
        in_specs=[
            pl.BlockSpec((N, 4 * CO), lambda b: (b, 0)),
            pl.BlockSpec((N, CO), lambda b: (b, 0)),
            pl.BlockSpec((1, CO), lambda b: (0, 0)),
            pl.BlockSpec((1, CO), lambda b: (0, 0)),
        ],
        out_specs=pl.BlockSpec((N, CO), lambda b: (b, 0)),
        out_shape=jax.ShapeDtypeStruct((BN, CO), jnp.float32),
    )(stats.reshape(BN, 4 * CO), b_arr, gn_gamma[None, :], gn_beta[None, :])

    return out.reshape(B, N, CO)
